# baseline (speedup 1.0000x reference)
import functools

import jax
import jax.numpy as jnp
from jax import lax
from jax.experimental import pallas as pl
from jax.experimental.pallas import tpu as pltpu
from jax.experimental.pallas import tpu_sc as plsc

N = 100000
E = 3200000
C = 64
NC = 2
NS = 16
L = 16
NW = NC * NS
N_PAD = 102400
EPW = E // NW
SLICE = N_PAD // NS
SUB = 1600

CHUNK1 = 20000
CHUNK2 = 2000
NCH1 = EPW // CHUNK1
NCH2 = EPW // CHUNK2

_MESH = plsc.VectorSubcoreMesh(
    core_axis_name="c", subcore_axis_name="s", num_cores=NC, num_subcores=NS)
_SC_PARAMS = pltpu.CompilerParams(needs_layout_passes=False)


def _fill(ref, n, value, dtype, base=0):
    vec = jnp.full((L,), value, dtype)

    def body(i, _):
        ref[pl.ds(base + i * L, L)] = vec
        return 0

    lax.fori_loop(0, n // L, body, 0)


@functools.partial(
    pl.kernel,
    out_type=jax.ShapeDtypeStruct((NC * N_PAD,), jnp.float32),
    mesh=_MESH,
    scratch_types=[
        pltpu.VMEM((CHUNK1,), jnp.int32),
        pltpu.VMEM((CHUNK1,), jnp.int32),
        pltpu.VMEM((CHUNK1,), jnp.float32),
        pltpu.VMEM((SLICE,), jnp.float32),
        pltpu.VMEM_SHARED((N_PAD,), jnp.float32),
        pltpu.SemaphoreType.DMA,
        pltpu.SemaphoreType.DMA,
        pltpu.SemaphoreType.DMA,
        pltpu.SemaphoreType.DMA,
    ],
    compiler_params=_SC_PARAMS,
)
def _counts_kernel(src_hbm, out_hbm, src_a, src_b, ones_v, zeros_v, acc_sh,
                   sem_a, sem_b, ssem_a, ssem_b):
    c = lax.axis_index("c")
    s = lax.axis_index("s")
    wid = s * NC + c
    base = wid * EPW
    bufs = [(src_a, sem_a, ssem_a), (src_b, sem_b, ssem_b)]

    def _chunk(k):
        return src_hbm.at[pl.ds(base + k * CHUNK1, CHUNK1)]

    def _scat(k):
        buf, _, ssem = bufs[k % 2]
        return pltpu.make_async_copy(ones_v, acc_sh.at[buf], ssem)

    pltpu.async_copy(_chunk(0), src_a, sem_a)
    _fill(ones_v, CHUNK1, 1.0, jnp.float32)
    _fill(zeros_v, SLICE, 0.0, jnp.float32)
    pltpu.sync_copy(zeros_v, acc_sh.at[pl.ds(s * SLICE, SLICE)])
    plsc.subcore_barrier()

    for k in range(NCH1):
        buf, sem, ssem = bufs[k % 2]
        pltpu.make_async_copy(_chunk(k), buf, sem).wait()
        if k - 1 >= 0:
            _scat(k - 1).wait()
        if k + 1 < NCH1:
            nbuf, nsem, _ = bufs[(k + 1) % 2]
            pltpu.async_copy(_chunk(k + 1), nbuf, nsem)
        pltpu.async_copy(ones_v, acc_sh.at[buf], ssem, add=True)

    _scat(NCH1 - 1).wait()
    plsc.subcore_barrier()
    pltpu.sync_copy(acc_sh.at[pl.ds(s * SLICE, SLICE)],
                    out_hbm.at[pl.ds(c * N_PAD + s * SLICE, SLICE)])


def _rsqrt_newton(x):
    i = plsc.bitcast(x, jnp.int32)
    i = jnp.int32(0x5F3759DF) - (i >> 1)
    y = plsc.bitcast(i, jnp.float32)
    xh = x * 0.5
    for _ in range(3):
        y = y * (1.5 - xh * y * y)
    return y


@functools.partial(
    pl.kernel,
    out_type=(
        jax.ShapeDtypeStruct((NC, N_PAD), jnp.float32),
        jax.ShapeDtypeStruct((NC * N_PAD,), jnp.float32),
    ),
    mesh=_MESH,
    scratch_types=[
        pltpu.VMEM((N, ), jnp.float32),
        pltpu.VMEM((CHUNK2,), jnp.int32),
        pltpu.VMEM((CHUNK2,), jnp.int32),
        pltpu.VMEM((CHUNK2,), jnp.int32),
        pltpu.VMEM((CHUNK2,), jnp.int32),
        pltpu.VMEM((CHUNK2,), jnp.int32),
        pltpu.VMEM((CHUNK2,), jnp.int32),
        pltpu.VMEM((CHUNK2,), jnp.float32),
        pltpu.VMEM((CHUNK2,), jnp.float32),
        pltpu.VMEM((CHUNK2,), jnp.float32),
        pltpu.VMEM_SHARED((N_PAD,), jnp.float32),
        pltpu.SemaphoreType.DMA,
        pltpu.SemaphoreType.DMA,
        pltpu.SemaphoreType.DMA,
        pltpu.SemaphoreType.DMA,
        pltpu.SemaphoreType.DMA,
        pltpu.SemaphoreType.DMA,
        pltpu.SemaphoreType.DMA,
        pltpu.SemaphoreType.DMA,
        pltpu.SemaphoreType.DMA,
        pltpu.SemaphoreType.DMA,
    ],
    compiler_params=_SC_PARAMS,
)
def _agg_kernel(counts_hbm, feat_hbm, src_hbm, dst_hbm, out_hbm, x_hbm,
                x_v, src_a, src_b, src_c, src_d, dst_a, dst_b,
                val_a, val_b, val_c,
                acc_sh, lsem_a, lsem_b, lsem_c, lsem_d,
                dsem_a, dsem_b, ssem_a, ssem_b, ssem_c, psem):
    c = lax.axis_index("c")
    s = lax.axis_index("s")
    wid = s * NC + c
    base = wid * EPW
    off = s * SLICE

    pltpu.async_copy(src_hbm.at[pl.ds(base, CHUNK2)], src_a, lsem_a)
    pltpu.async_copy(dst_hbm.at[pl.ds(base, CHUNK2)], dst_a, dsem_a)

    cp0 = pltpu.make_async_copy(
        counts_hbm.at[pl.ds(off, SLICE)], x_v.at[pl.ds(0, SLICE)], psem)
    cp1 = pltpu.make_async_copy(
        counts_hbm.at[pl.ds(N_PAD + off, SLICE)],
        x_v.at[pl.ds(SLICE, SLICE)], psem)
    cp2 = pltpu.make_async_copy(
        feat_hbm.at[pl.ds(off, SLICE)], x_v.at[pl.ds(2 * SLICE, SLICE)], psem)
    cp0.start()
    cp1.start()
    cp2.start()
    _fill(x_v, SLICE, 0.0, jnp.float32, base=4 * SLICE)
    pltpu.sync_copy(x_v.at[pl.ds(4 * SLICE, SLICE)],
                    acc_sh.at[pl.ds(off, SLICE)])
    cp0.wait()
    cp1.wait()
    cp2.wait()

    def xbody(i, _):
        for u in range(4):
            o = i * (4 * L) + u * L
            cnt = x_v[pl.ds(o, L)] + x_v[pl.ds(SLICE + o, L)]
            d = _rsqrt_newton(cnt)
            x_v[pl.ds(3 * SLICE + o, L)] = x_v[pl.ds(2 * SLICE + o, L)] * d
        return 0

    lax.fori_loop(0, SLICE // (4 * L), xbody, 0)
    pltpu.sync_copy(x_v.at[pl.ds(3 * SLICE, SLICE)],
                    x_hbm.at[pl.ds(c * N_PAD + off, SLICE)])
    plsc.subcore_barrier()

    pltpu.sync_copy(x_hbm.at[pl.ds(c * N_PAD, N)], x_v)

    srcs = [src_a, src_b, src_c, src_d]
    lsems = [lsem_a, lsem_b, lsem_c, lsem_d]
    dsts = [dst_a, dst_b]
    dsems = [dsem_a, dsem_b]
    vals = [val_a, val_b, val_c]
    ssems = [ssem_a, ssem_b, ssem_c]

    def _src(k):
        return src_hbm.at[pl.ds(base + k * CHUNK2, CHUNK2)]

    def _dst(k):
        return dst_hbm.at[pl.ds(base + k * CHUNK2, CHUNK2)]

    def _scat(k):
        return pltpu.make_async_copy(vals[k % 3], acc_sh.at[srcs[k % 4]],
                                     ssems[k % 3])

    for k in range(NCH2):
        sb, db, vb = srcs[k % 4], dsts[k % 2], vals[k % 3]
        pltpu.make_async_copy(_src(k), sb, lsems[k % 4]).wait()
        pltpu.make_async_copy(_dst(k), db, dsems[k % 2]).wait()
        if k - 3 >= 0:
            _scat(k - 3).wait()
        if k + 1 < NCH2:
            pltpu.async_copy(_src(k + 1), srcs[(k + 1) % 4], lsems[(k + 1) % 4])
            pltpu.async_copy(_dst(k + 1), dsts[(k + 1) % 2], dsems[(k + 1) % 2])

        def gather(i, _):
            gb = i * (25 * L)
            for u in range(25):
                o = gb + u * L
                idx = db[pl.ds(o, L)]
                vb[pl.ds(o, L)] = plsc.load_gather(x_v, [idx])
            return 0

        lax.fori_loop(0, CHUNK2 // (25 * L), gather, 0)
        pltpu.async_copy(vb, acc_sh.at[sb], ssems[k % 3], add=True)

    _scat(NCH2 - 3).wait()
    _scat(NCH2 - 2).wait()
    _scat(NCH2 - 1).wait()

    plsc.subcore_barrier()
    pltpu.sync_copy(acc_sh.at[pl.ds(s * SLICE, SLICE)],
                    out_hbm.at[c, pl.ds(s * SLICE, SLICE)])


BLK = 16384
GRID = (N + BLK - 1) // BLK


def _matvec_body(cnt_ref, s_ref, w_ref, b_ref, out_ref):
    j = pl.program_id(0)
    last = pl.num_programs(0) - 1

    @pl.when(j == 0)
    def _():
        out_ref[...] = jnp.zeros_like(out_ref)

    @pl.when(j != last)
    def _():
        d = lax.rsqrt(cnt_ref[0:1, :] + cnt_ref[1:2, :])
        agg = (s_ref[0:1, :] + s_ref[1:2, :]) * d
        part = lax.dot_general(w_ref[...], agg, (((1,), (1,)), ((), ())),
                               preferred_element_type=jnp.float32)
        out_ref[0, :] += part[:, 0]

    @pl.when(j == last)
    def _():
        cols = j * BLK + lax.broadcasted_iota(jnp.int32, (1, BLK), 1)
        mask = cols < N
        d = lax.rsqrt(cnt_ref[0:1, :] + cnt_ref[1:2, :])
        agg = jnp.where(mask, (s_ref[0:1, :] + s_ref[1:2, :]) * d, 0.0)
        w = jnp.where(mask, w_ref[...], 0.0)
        part = lax.dot_general(w, agg, (((1,), (1,)), ((), ())),
                               preferred_element_type=jnp.float32)
        out_ref[0, :] += part[:, 0] + b_ref[...]
        out_ref[1, :] = b_ref[...]


def _matvec(counts_part, s_part, W, b):
    return pl.pallas_call(
        _matvec_body,
        grid=(GRID,),
        in_specs=[
            pl.BlockSpec((NC, BLK), lambda j: (0, j)),
            pl.BlockSpec((NC, BLK), lambda j: (0, j)),
            pl.BlockSpec((C, BLK), lambda j: (0, j)),
            pl.BlockSpec((C,), lambda j: (0,)),
        ],
        out_specs=pl.BlockSpec((2, C), lambda j: (0, 0)),
        out_shape=jax.ShapeDtypeStruct((2, C), jnp.float32),
    )(counts_part, s_part, W, b)


def kernel(nodes_feat_list, edges_index_list, graph_label_list, W, b):
    del graph_label_list
    src = edges_index_list[0, 0, :].astype(jnp.int32)
    dst = edges_index_list[0, 1, :].astype(jnp.int32)
    feat_pad = jnp.pad(nodes_feat_list[0, :, 0], (0, N_PAD - N))

    counts_flat = _counts_kernel(src)
    s_part, _ = _agg_kernel(counts_flat, feat_pad, src, dst)
    out2 = _matvec(counts_flat.reshape(NC, N_PAD), s_part, W, b)
    return out2[:, None, :]


# speedup vs baseline: 1.1475x; 1.1475x over previous
import functools

import jax
import jax.numpy as jnp
from jax import lax
from jax.experimental import pallas as pl
from jax.experimental.pallas import tpu as pltpu
from jax.experimental.pallas import tpu_sc as plsc

N = 100000
E = 3200000
C = 64
NC = 2
NS = 16
L = 16
NW = NC * NS
N_PAD = 102400
EPW = E // NW
SLICE = N_PAD // NS
SUB = 1600

CHUNK1 = 20000
CHUNK2 = 2000
NCH1 = EPW // CHUNK1
NCH2 = EPW // CHUNK2

_MESH = plsc.VectorSubcoreMesh(
    core_axis_name="c", subcore_axis_name="s", num_cores=NC, num_subcores=NS)
_SC_PARAMS = pltpu.CompilerParams(needs_layout_passes=False)


def _fill(ref, n, value, dtype, base=0):
    vec = jnp.full((L,), value, dtype)

    def body(i, _):
        ref[pl.ds(base + i * L, L)] = vec
        return 0

    lax.fori_loop(0, n // L, body, 0)


@functools.partial(
    pl.kernel,
    out_type=jax.ShapeDtypeStruct((NC * N_PAD,), jnp.float32),
    mesh=_MESH,
    scratch_types=[
        pltpu.VMEM((CHUNK1,), jnp.int32),
        pltpu.VMEM((CHUNK1,), jnp.int32),
        pltpu.VMEM((CHUNK1,), jnp.float32),
        pltpu.VMEM((SLICE,), jnp.float32),
        pltpu.VMEM_SHARED((N_PAD,), jnp.float32),
        pltpu.SemaphoreType.DMA,
        pltpu.SemaphoreType.DMA,
        pltpu.SemaphoreType.DMA,
        pltpu.SemaphoreType.DMA,
    ],
    compiler_params=_SC_PARAMS,
)
def _counts_kernel(src_hbm, out_hbm, src_a, src_b, ones_v, zeros_v, acc_sh,
                   sem_a, sem_b, ssem_a, ssem_b):
    c = lax.axis_index("c")
    s = lax.axis_index("s")
    wid = s * NC + c
    base = wid * EPW
    bufs = [(src_a, sem_a, ssem_a), (src_b, sem_b, ssem_b)]

    def _chunk(k):
        return src_hbm.at[pl.ds(base + k * CHUNK1, CHUNK1)]

    def _scat(k):
        buf, _, ssem = bufs[k % 2]
        return pltpu.make_async_copy(ones_v, acc_sh.at[buf], ssem)

    pltpu.async_copy(_chunk(0), src_a, sem_a)
    _fill(ones_v, CHUNK1, 1.0, jnp.float32)
    _fill(zeros_v, SLICE, 0.0, jnp.float32)
    pltpu.sync_copy(zeros_v, acc_sh.at[pl.ds(s * SLICE, SLICE)])
    plsc.subcore_barrier()

    for k in range(NCH1):
        buf, sem, ssem = bufs[k % 2]
        pltpu.make_async_copy(_chunk(k), buf, sem).wait()
        if k - 1 >= 0:
            _scat(k - 1).wait()
        if k + 1 < NCH1:
            nbuf, nsem, _ = bufs[(k + 1) % 2]
            pltpu.async_copy(_chunk(k + 1), nbuf, nsem)
        pltpu.async_copy(ones_v, acc_sh.at[buf], ssem, add=True)

    _scat(NCH1 - 1).wait()
    plsc.subcore_barrier()
    pltpu.sync_copy(acc_sh.at[pl.ds(s * SLICE, SLICE)],
                    out_hbm.at[pl.ds(c * N_PAD + s * SLICE, SLICE)])


def _rsqrt_newton(x):
    i = plsc.bitcast(x, jnp.int32)
    i = jnp.int32(0x5F3759DF) - (i >> 1)
    y = plsc.bitcast(i, jnp.float32)
    xh = x * 0.5
    for _ in range(3):
        y = y * (1.5 - xh * y * y)
    return y


@functools.partial(
    pl.kernel,
    out_type=(
        jax.ShapeDtypeStruct((NC, N_PAD), jnp.float32),
        jax.ShapeDtypeStruct((NC * N_PAD,), jnp.float32),
    ),
    mesh=_MESH,
    scratch_types=[
        pltpu.VMEM((N, ), jnp.float32),
        pltpu.VMEM((CHUNK2,), jnp.int32),
        pltpu.VMEM((CHUNK2,), jnp.int32),
        pltpu.VMEM((CHUNK2,), jnp.int32),
        pltpu.VMEM((CHUNK2,), jnp.int32),
        pltpu.VMEM((CHUNK2,), jnp.int32),
        pltpu.VMEM((CHUNK2,), jnp.int32),
        pltpu.VMEM((CHUNK2,), jnp.float32),
        pltpu.VMEM((CHUNK2,), jnp.float32),
        pltpu.VMEM((CHUNK2,), jnp.float32),
        pltpu.VMEM_SHARED((N_PAD,), jnp.float32),
        pltpu.SemaphoreType.DMA,
        pltpu.SemaphoreType.DMA,
        pltpu.SemaphoreType.DMA,
        pltpu.SemaphoreType.DMA,
        pltpu.SemaphoreType.DMA,
        pltpu.SemaphoreType.DMA,
        pltpu.SemaphoreType.DMA,
        pltpu.SemaphoreType.DMA,
        pltpu.SemaphoreType.DMA,
        pltpu.SemaphoreType.DMA,
    ],
    compiler_params=_SC_PARAMS,
)
def _agg_kernel(counts_hbm, feat_hbm, src_hbm, dst_hbm, out_hbm, x_hbm,
                x_v, src_a, src_b, src_c, src_d, dst_a, dst_b,
                val_a, val_b, val_c,
                acc_sh, lsem_a, lsem_b, lsem_c, lsem_d,
                dsem_a, dsem_b, ssem_a, ssem_b, ssem_c, psem):
    c = lax.axis_index("c")
    s = lax.axis_index("s")
    wid = s * NC + c
    base = wid * EPW
    off = s * SLICE

    pltpu.async_copy(src_hbm.at[pl.ds(base, CHUNK2)], src_a, lsem_a)
    pltpu.async_copy(dst_hbm.at[pl.ds(base, CHUNK2)], dst_a, dsem_a)

    cp0 = pltpu.make_async_copy(
        counts_hbm.at[pl.ds(off, SLICE)], x_v.at[pl.ds(0, SLICE)], psem)
    cp1 = pltpu.make_async_copy(
        counts_hbm.at[pl.ds(N_PAD + off, SLICE)],
        x_v.at[pl.ds(SLICE, SLICE)], psem)
    cp2 = pltpu.make_async_copy(
        feat_hbm.at[pl.ds(off, SLICE)], x_v.at[pl.ds(2 * SLICE, SLICE)], psem)
    cp0.start()
    cp1.start()
    cp2.start()
    _fill(x_v, SLICE, 0.0, jnp.float32, base=4 * SLICE)
    pltpu.sync_copy(x_v.at[pl.ds(4 * SLICE, SLICE)],
                    acc_sh.at[pl.ds(off, SLICE)])
    cp0.wait()
    cp1.wait()
    cp2.wait()

    def xbody(i, _):
        for u in range(4):
            o = i * (4 * L) + u * L
            cnt = x_v[pl.ds(o, L)] + x_v[pl.ds(SLICE + o, L)]
            d = _rsqrt_newton(cnt)
            x_v[pl.ds(3 * SLICE + o, L)] = x_v[pl.ds(2 * SLICE + o, L)] * d
        return 0

    lax.fori_loop(0, SLICE // (4 * L), xbody, 0)
    pltpu.sync_copy(x_v.at[pl.ds(3 * SLICE, SLICE)],
                    x_hbm.at[pl.ds(c * N_PAD + off, SLICE)])
    plsc.subcore_barrier()

    pltpu.sync_copy(x_hbm.at[pl.ds(c * N_PAD, N)], x_v)

    srcs = [src_a, src_b, src_c, src_d]
    lsems = [lsem_a, lsem_b, lsem_c, lsem_d]
    dsts = [dst_a, dst_b]
    dsems = [dsem_a, dsem_b]
    vals = [val_a, val_b, val_c]
    ssems = [ssem_a, ssem_b, ssem_c]

    def _src(k):
        return src_hbm.at[pl.ds(base + k * CHUNK2, CHUNK2)]

    def _dst(k):
        return dst_hbm.at[pl.ds(base + k * CHUNK2, CHUNK2)]

    def _scat(k):
        return pltpu.make_async_copy(vals[k % 3], acc_sh.at[srcs[k % 4]],
                                     ssems[k % 3])

    for k in range(NCH2):
        sb, db, vb = srcs[k % 4], dsts[k % 2], vals[k % 3]
        pltpu.make_async_copy(_src(k), sb, lsems[k % 4]).wait()
        pltpu.make_async_copy(_dst(k), db, dsems[k % 2]).wait()
        if k - 3 >= 0:
            _scat(k - 3).wait()
        if k + 1 < NCH2:
            pltpu.async_copy(_src(k + 1), srcs[(k + 1) % 4], lsems[(k + 1) % 4])
            pltpu.async_copy(_dst(k + 1), dsts[(k + 1) % 2], dsems[(k + 1) % 2])

        def gather(i, _):
            gb = i * (5 * L)
            for u in range(5):
                o = gb + u * L
                idx = db[pl.ds(o, L)]
                vb[pl.ds(o, L)] = plsc.load_gather(x_v, [idx])
            return 0

        lax.fori_loop(0, CHUNK2 // (5 * L), gather, 0)
        pltpu.async_copy(vb, acc_sh.at[sb], ssems[k % 3], add=True)

    _scat(NCH2 - 3).wait()
    _scat(NCH2 - 2).wait()
    _scat(NCH2 - 1).wait()

    plsc.subcore_barrier()
    pltpu.sync_copy(acc_sh.at[pl.ds(s * SLICE, SLICE)],
                    out_hbm.at[c, pl.ds(s * SLICE, SLICE)])


BLK = 16384
GRID = (N + BLK - 1) // BLK


def _matvec_body(cnt_ref, s_ref, w_ref, b_ref, out_ref):
    j = pl.program_id(0)
    last = pl.num_programs(0) - 1

    @pl.when(j == 0)
    def _():
        out_ref[...] = jnp.zeros_like(out_ref)

    @pl.when(j != last)
    def _():
        d = lax.rsqrt(cnt_ref[0:1, :] + cnt_ref[1:2, :])
        agg = (s_ref[0:1, :] + s_ref[1:2, :]) * d
        part = lax.dot_general(w_ref[...], agg, (((1,), (1,)), ((), ())),
                               preferred_element_type=jnp.float32)
        out_ref[0, :] += part[:, 0]

    @pl.when(j == last)
    def _():
        cols = j * BLK + lax.broadcasted_iota(jnp.int32, (1, BLK), 1)
        mask = cols < N
        d = lax.rsqrt(cnt_ref[0:1, :] + cnt_ref[1:2, :])
        agg = jnp.where(mask, (s_ref[0:1, :] + s_ref[1:2, :]) * d, 0.0)
        w = jnp.where(mask, w_ref[...], 0.0)
        part = lax.dot_general(w, agg, (((1,), (1,)), ((), ())),
                               preferred_element_type=jnp.float32)
        out_ref[0, :] += part[:, 0] + b_ref[...]
        out_ref[1, :] = b_ref[...]


def _matvec(counts_part, s_part, W, b):
    return pl.pallas_call(
        _matvec_body,
        grid=(GRID,),
        in_specs=[
            pl.BlockSpec((NC, BLK), lambda j: (0, j)),
            pl.BlockSpec((NC, BLK), lambda j: (0, j)),
            pl.BlockSpec((C, BLK), lambda j: (0, j)),
            pl.BlockSpec((C,), lambda j: (0,)),
        ],
        out_specs=pl.BlockSpec((2, C), lambda j: (0, 0)),
        out_shape=jax.ShapeDtypeStruct((2, C), jnp.float32),
    )(counts_part, s_part, W, b)


def kernel(nodes_feat_list, edges_index_list, graph_label_list, W, b):
    del graph_label_list
    src = edges_index_list[0, 0, :].astype(jnp.int32)
    dst = edges_index_list[0, 1, :].astype(jnp.int32)
    feat_pad = jnp.pad(nodes_feat_list[0, :, 0], (0, N_PAD - N))

    counts_flat = _counts_kernel(src)
    s_part, _ = _agg_kernel(counts_flat, feat_pad, src, dst)
    out2 = _matvec(counts_flat.reshape(NC, N_PAD), s_part, W, b)
    return out2[:, None, :]


# speedup vs baseline: 1.1611x; 1.0118x over previous
import functools

import jax
import jax.numpy as jnp
from jax import lax
from jax.experimental import pallas as pl
from jax.experimental.pallas import tpu as pltpu
from jax.experimental.pallas import tpu_sc as plsc

N = 100000
E = 3200000
C = 64
NC = 2
NS = 16
L = 16
NW = NC * NS
N_PAD = 102400
EPW = E // NW
SLICE = N_PAD // NS
SUB = 1600

CHUNK1 = 20000
CHUNK2 = 2000
NCH1 = EPW // CHUNK1
NCH2 = EPW // CHUNK2

_MESH = plsc.VectorSubcoreMesh(
    core_axis_name="c", subcore_axis_name="s", num_cores=NC, num_subcores=NS)
_SC_PARAMS = pltpu.CompilerParams(needs_layout_passes=False)


def _fill(ref, n, value, dtype, base=0):
    vec = jnp.full((L,), value, dtype)

    def body(i, _):
        ref[pl.ds(base + i * L, L)] = vec
        return 0

    lax.fori_loop(0, n // L, body, 0)


@functools.partial(
    pl.kernel,
    out_type=jax.ShapeDtypeStruct((NC * N_PAD,), jnp.float32),
    mesh=_MESH,
    scratch_types=[
        pltpu.VMEM((CHUNK1,), jnp.int32),
        pltpu.VMEM((CHUNK1,), jnp.int32),
        pltpu.VMEM((CHUNK1,), jnp.float32),
        pltpu.VMEM((SLICE,), jnp.float32),
        pltpu.VMEM_SHARED((N_PAD,), jnp.float32),
        pltpu.SemaphoreType.DMA,
        pltpu.SemaphoreType.DMA,
        pltpu.SemaphoreType.DMA,
        pltpu.SemaphoreType.DMA,
    ],
    compiler_params=_SC_PARAMS,
)
def _counts_kernel(src_hbm, out_hbm, src_a, src_b, ones_v, zeros_v, acc_sh,
                   sem_a, sem_b, ssem_a, ssem_b):
    c = lax.axis_index("c")
    s = lax.axis_index("s")
    wid = s * NC + c
    base = wid * EPW
    bufs = [(src_a, sem_a, ssem_a), (src_b, sem_b, ssem_b)]

    def _chunk(k):
        return src_hbm.at[pl.ds(base + k * CHUNK1, CHUNK1)]

    def _scat(k):
        buf, _, ssem = bufs[k % 2]
        return pltpu.make_async_copy(ones_v, acc_sh.at[buf], ssem)

    pltpu.async_copy(_chunk(0), src_a, sem_a)
    _fill(ones_v, CHUNK1, 1.0, jnp.float32)
    _fill(zeros_v, SLICE, 0.0, jnp.float32)
    pltpu.sync_copy(zeros_v, acc_sh.at[pl.ds(s * SLICE, SLICE)])
    plsc.subcore_barrier()

    for k in range(NCH1):
        buf, sem, ssem = bufs[k % 2]
        pltpu.make_async_copy(_chunk(k), buf, sem).wait()
        if k - 1 >= 0:
            _scat(k - 1).wait()
        if k + 1 < NCH1:
            nbuf, nsem, _ = bufs[(k + 1) % 2]
            pltpu.async_copy(_chunk(k + 1), nbuf, nsem)
        pltpu.async_copy(ones_v, acc_sh.at[buf], ssem, add=True)

    _scat(NCH1 - 1).wait()
    plsc.subcore_barrier()
    pltpu.sync_copy(acc_sh.at[pl.ds(s * SLICE, SLICE)],
                    out_hbm.at[pl.ds(c * N_PAD + s * SLICE, SLICE)])


def _rsqrt_newton(x):
    i = plsc.bitcast(x, jnp.int32)
    i = jnp.int32(0x5F3759DF) - (i >> 1)
    y = plsc.bitcast(i, jnp.float32)
    xh = x * 0.5
    for _ in range(3):
        y = y * (1.5 - xh * y * y)
    return y


@functools.partial(
    pl.kernel,
    out_type=(
        jax.ShapeDtypeStruct((NC, N_PAD), jnp.float32),
        jax.ShapeDtypeStruct((NC * N_PAD,), jnp.float32),
    ),
    mesh=_MESH,
    scratch_types=[
        pltpu.VMEM((N_PAD, ), jnp.float32),
        pltpu.VMEM((CHUNK2,), jnp.int32),
        pltpu.VMEM((CHUNK2,), jnp.int32),
        pltpu.VMEM((CHUNK2,), jnp.int32),
        pltpu.VMEM((CHUNK2,), jnp.int32),
        pltpu.VMEM((CHUNK2,), jnp.int32),
        pltpu.VMEM((CHUNK2,), jnp.int32),
        pltpu.VMEM((CHUNK2,), jnp.float32),
        pltpu.VMEM((CHUNK2,), jnp.float32),
        pltpu.VMEM((CHUNK2,), jnp.float32),
        pltpu.VMEM_SHARED((N_PAD,), jnp.float32),
        pltpu.SemaphoreType.DMA,
        pltpu.SemaphoreType.DMA,
        pltpu.SemaphoreType.DMA,
        pltpu.SemaphoreType.DMA,
        pltpu.SemaphoreType.DMA,
        pltpu.SemaphoreType.DMA,
        pltpu.SemaphoreType.DMA,
        pltpu.SemaphoreType.DMA,
        pltpu.SemaphoreType.DMA,
        pltpu.SemaphoreType.DMA,
    ],
    compiler_params=_SC_PARAMS,
)
def _agg_kernel(counts_hbm, feat_hbm, src_hbm, dst_hbm, out_hbm, x_hbm,
                x_v, src_a, src_b, src_c, src_d, dst_a, dst_b,
                val_a, val_b, val_c,
                acc_sh, lsem_a, lsem_b, lsem_c, lsem_d,
                dsem_a, dsem_b, ssem_a, ssem_b, ssem_c, psem):
    c = lax.axis_index("c")
    s = lax.axis_index("s")
    wid = s * NC + c
    base = wid * EPW
    off = s * SLICE

    pltpu.async_copy(src_hbm.at[pl.ds(base, CHUNK2)], src_a, lsem_a)
    pltpu.async_copy(dst_hbm.at[pl.ds(base, CHUNK2)], dst_a, dsem_a)

    cp0 = pltpu.make_async_copy(
        counts_hbm.at[pl.ds(off, SLICE)], x_v.at[pl.ds(0, SLICE)], psem)
    cp1 = pltpu.make_async_copy(
        counts_hbm.at[pl.ds(N_PAD + off, SLICE)],
        x_v.at[pl.ds(SLICE, SLICE)], psem)
    cp2 = pltpu.make_async_copy(
        feat_hbm.at[pl.ds(off, SLICE)], x_v.at[pl.ds(2 * SLICE, SLICE)], psem)
    cp0.start()
    cp1.start()
    cp2.start()
    _fill(x_v, SLICE, 0.0, jnp.float32, base=4 * SLICE)
    pltpu.sync_copy(x_v.at[pl.ds(4 * SLICE, SLICE)],
                    acc_sh.at[pl.ds(off, SLICE)])
    cp0.wait()
    cp1.wait()
    cp2.wait()

    def xbody(i, _):
        for u in range(4):
            o = i * (4 * L) + u * L
            cnt = x_v[pl.ds(o, L)] + x_v[pl.ds(SLICE + o, L)]
            d = _rsqrt_newton(cnt)
            x_v[pl.ds(3 * SLICE + o, L)] = x_v[pl.ds(2 * SLICE + o, L)] * d
        return 0

    lax.fori_loop(0, SLICE // (4 * L), xbody, 0)
    pltpu.sync_copy(x_v.at[pl.ds(3 * SLICE, SLICE)],
                    x_hbm.at[pl.ds(c * N_PAD + off, SLICE)])
    plsc.subcore_barrier()

    def _xpiece(q):
        pc = lax.rem(s + q, NS)
        return pltpu.make_async_copy(
            x_hbm.at[pl.ds(c * N_PAD + pc * SLICE, SLICE)],
            x_v.at[pl.ds(pc * SLICE, SLICE)], psem)

    for q in range(NS):
        _xpiece(q).start()
    for q in range(NS):
        _xpiece(q).wait()

    srcs = [src_a, src_b, src_c, src_d]
    lsems = [lsem_a, lsem_b, lsem_c, lsem_d]
    dsts = [dst_a, dst_b]
    dsems = [dsem_a, dsem_b]
    vals = [val_a, val_b, val_c]
    ssems = [ssem_a, ssem_b, ssem_c]

    def _src(k):
        return src_hbm.at[pl.ds(base + k * CHUNK2, CHUNK2)]

    def _dst(k):
        return dst_hbm.at[pl.ds(base + k * CHUNK2, CHUNK2)]

    def _scat(k):
        return pltpu.make_async_copy(vals[k % 3], acc_sh.at[srcs[k % 4]],
                                     ssems[k % 3])

    for k in range(NCH2):
        sb, db, vb = srcs[k % 4], dsts[k % 2], vals[k % 3]
        pltpu.make_async_copy(_src(k), sb, lsems[k % 4]).wait()
        pltpu.make_async_copy(_dst(k), db, dsems[k % 2]).wait()
        if k - 3 >= 0:
            _scat(k - 3).wait()
        if k + 1 < NCH2:
            pltpu.async_copy(_src(k + 1), srcs[(k + 1) % 4], lsems[(k + 1) % 4])
            pltpu.async_copy(_dst(k + 1), dsts[(k + 1) % 2], dsems[(k + 1) % 2])

        def gather(i, _):
            gb = i * (5 * L)
            for u in range(5):
                o = gb + u * L
                idx = db[pl.ds(o, L)]
                vb[pl.ds(o, L)] = plsc.load_gather(x_v, [idx])
            return 0

        lax.fori_loop(0, CHUNK2 // (5 * L), gather, 0)
        pltpu.async_copy(vb, acc_sh.at[sb], ssems[k % 3], add=True)

    _scat(NCH2 - 3).wait()
    _scat(NCH2 - 2).wait()
    _scat(NCH2 - 1).wait()

    plsc.subcore_barrier()
    pltpu.sync_copy(acc_sh.at[pl.ds(s * SLICE, SLICE)],
                    out_hbm.at[c, pl.ds(s * SLICE, SLICE)])


BLK = 16384
GRID = (N + BLK - 1) // BLK


def _matvec_body(cnt_ref, s_ref, w_ref, b_ref, out_ref):
    j = pl.program_id(0)
    last = pl.num_programs(0) - 1

    @pl.when(j == 0)
    def _():
        out_ref[...] = jnp.zeros_like(out_ref)

    @pl.when(j != last)
    def _():
        d = lax.rsqrt(cnt_ref[0:1, :] + cnt_ref[1:2, :])
        agg = (s_ref[0:1, :] + s_ref[1:2, :]) * d
        part = lax.dot_general(w_ref[...], agg, (((1,), (1,)), ((), ())),
                               preferred_element_type=jnp.float32)
        out_ref[0, :] += part[:, 0]

    @pl.when(j == last)
    def _():
        cols = j * BLK + lax.broadcasted_iota(jnp.int32, (1, BLK), 1)
        mask = cols < N
        d = lax.rsqrt(cnt_ref[0:1, :] + cnt_ref[1:2, :])
        agg = jnp.where(mask, (s_ref[0:1, :] + s_ref[1:2, :]) * d, 0.0)
        w = jnp.where(mask, w_ref[...], 0.0)
        part = lax.dot_general(w, agg, (((1,), (1,)), ((), ())),
                               preferred_element_type=jnp.float32)
        out_ref[0, :] += part[:, 0] + b_ref[...]
        out_ref[1, :] = b_ref[...]


def _matvec(counts_part, s_part, W, b):
    return pl.pallas_call(
        _matvec_body,
        grid=(GRID,),
        in_specs=[
            pl.BlockSpec((NC, BLK), lambda j: (0, j)),
            pl.BlockSpec((NC, BLK), lambda j: (0, j)),
            pl.BlockSpec((C, BLK), lambda j: (0, j)),
            pl.BlockSpec((C,), lambda j: (0,)),
        ],
        out_specs=pl.BlockSpec((2, C), lambda j: (0, 0)),
        out_shape=jax.ShapeDtypeStruct((2, C), jnp.float32),
    )(counts_part, s_part, W, b)


def kernel(nodes_feat_list, edges_index_list, graph_label_list, W, b):
    del graph_label_list
    src = edges_index_list[0, 0, :].astype(jnp.int32)
    dst = edges_index_list[0, 1, :].astype(jnp.int32)
    feat_pad = jnp.pad(nodes_feat_list[0, :, 0], (0, N_PAD - N))

    counts_flat = _counts_kernel(src)
    s_part, _ = _agg_kernel(counts_flat, feat_pad, src, dst)
    out2 = _matvec(counts_flat.reshape(NC, N_PAD), s_part, W, b)
    return out2[:, None, :]
